# Initial kernel scaffold; baseline (speedup 1.0000x reference)
import functools

import jax
import jax.numpy as jnp
from jax import lax
from jax.experimental import pallas as pl
from jax.experimental.pallas import tpu as pltpu
import jax.experimental.pallas.tpu_sc as plsc

N = 320000
D = 128
S = 2048
NC = 2
NS = 16
NW = NC * NS
SPT = S // NW
C = 128
ACC_ROWS = SPT + 8
LANES = 16


def _tile_body(nodes_hbm, batch_hbm, bounds_hbm, out_hbm,
               rows_v, ids_v, lidx_v, bnd_v, acc_v):
    wid = lax.axis_index("s") * NC + lax.axis_index("c")
    seg_base = wid * SPT

    pltpu.sync_copy(bounds_hbm.at[pl.ds(wid, 1)], bnd_v)
    lanes = lax.iota(jnp.int32, 16)
    bvec = bnd_v[0, :]
    start = jnp.max(jnp.where(lanes == 0, bvec, 0))
    end = jnp.max(jnp.where(lanes == 1, bvec, 0))

    zz = jnp.zeros((LANES,), jnp.float32)

    def zero_row(i, carry):
        for j in range(D // LANES):
            acc_v[i, pl.ds(j * LANES, LANES)] = zz
        return carry

    lax.fori_loop(0, SPT, zero_row, 0)

    astart = jnp.bitwise_and(start, jnp.int32(-8))
    nchunks = lax.shift_right_arithmetic(end - astart + (C - 1), 7)

    def chunk(k, carry):
        nominal = astart + k * C
        base = jnp.minimum(nominal, N - C)
        lo = jnp.maximum(start, nominal)
        hi = jnp.minimum(end, nominal + C)
        pltpu.sync_copy(batch_hbm.at[pl.ds(base, C)], ids_v)
        pltpu.sync_copy(nodes_hbm.at[pl.ds(base, C)], rows_v)
        for g in range(C // LANES):
            rg = base + (g * LANES) + lanes
            idv = ids_v[pl.ds(g * LANES, LANES)]
            keep = ((rg >= lo) & (rg < hi)
                    & (idv >= seg_base) & (idv < seg_base + SPT))
            lidx_v[pl.ds(g * LANES, LANES)] = jnp.where(
                keep, idv - seg_base, SPT)
        pltpu.sync_copy(rows_v, acc_v.at[lidx_v], add=True)
        return carry

    lax.fori_loop(0, nchunks, chunk, 0)

    pltpu.sync_copy(acc_v.at[pl.ds(0, SPT)], out_hbm.at[pl.ds(seg_base, SPT)])


@functools.partial(
    pl.kernel,
    out_type=jax.ShapeDtypeStruct((S, D), jnp.float32),
    mesh=plsc.VectorSubcoreMesh(core_axis_name="c", subcore_axis_name="s"),
    scratch_types=[
        pltpu.VMEM((C, D), jnp.float32),
        pltpu.VMEM((C,), jnp.int32),
        pltpu.VMEM((C,), jnp.int32),
        pltpu.VMEM((1, 16), jnp.int32),
        pltpu.VMEM((ACC_ROWS, D), jnp.float32),
    ],
)
def _segment_sum_sc(nodes_hbm, batch_hbm, bounds_hbm, out_hbm,
                    rows_v, ids_v, lidx_v, bnd_v, acc_v):
    _tile_body(nodes_hbm, batch_hbm, bounds_hbm, out_hbm,
               rows_v, ids_v, lidx_v, bnd_v, acc_v)


def kernel(node_features, batch, ptr):
    edges = jnp.arange(0, S + 1, SPT, dtype=jnp.int32)
    starts = jnp.searchsorted(batch, edges, side="left").astype(jnp.int32)
    bounds = jnp.zeros((NW, 16), jnp.int32)
    bounds = bounds.at[:, 0].set(starts[:-1]).at[:, 1].set(starts[1:])
    return _segment_sum_sc(node_features, batch, bounds)


# speedup vs baseline: 3.8242x; 3.8242x over previous
import functools

import jax
import jax.numpy as jnp
from jax import lax
from jax.experimental import pallas as pl
from jax.experimental.pallas import tpu as pltpu
import jax.experimental.pallas.tpu_sc as plsc

N = 320000
D = 128
S = 2048
NC = 2
NS = 16
NW = NC * NS
SPT = S // NW
C = 128
ACC_ROWS = SPT + 8
LANES = 16


def _tile_body(nodes_hbm, batch_hbm, bounds_hbm, out_hbm,
               rows_v, ids_v, lidx_v, bnd_v, acc_sh):
    sid = lax.axis_index("s")
    wid = sid * NC + lax.axis_index("c")
    seg_base = wid * SPT
    region = sid * ACC_ROWS

    pltpu.sync_copy(bounds_hbm.at[pl.ds(wid, 1)], bnd_v)
    lanes = lax.iota(jnp.int32, 16)
    bvec = bnd_v[0, :]
    start = bvec[0]
    end = bvec[1]

    zz = jnp.zeros((LANES,), jnp.float32)

    def zero_row(i, carry):
        for j in range(D // LANES):
            rows_v[i, pl.ds(j * LANES, LANES)] = zz
        return carry

    lax.fori_loop(0, ACC_ROWS, zero_row, 0)
    pltpu.sync_copy(rows_v.at[pl.ds(0, ACC_ROWS)],
                    acc_sh.at[pl.ds(region, ACC_ROWS)])

    astart = jnp.bitwise_and(start, jnp.int32(-8))
    nchunks = lax.shift_right_arithmetic(end - astart + (C - 1), 7)

    def chunk(k, carry):
        nominal = astart + k * C
        base = jnp.minimum(nominal, N - C)
        base = pl.multiple_of(base, 8)
        lo = jnp.maximum(start, nominal)
        hi = jnp.minimum(end, nominal + C)
        pltpu.sync_copy(batch_hbm.at[pl.ds(base, C)], ids_v)
        pltpu.sync_copy(nodes_hbm.at[pl.ds(base, C)], rows_v)
        for g in range(C // LANES):
            rg = base + (g * LANES) + lanes
            idv = ids_v[pl.ds(g * LANES, LANES)]
            keep = ((rg >= lo) & (rg < hi)
                    & (idv >= seg_base) & (idv < seg_base + SPT))
            lidx_v[pl.ds(g * LANES, LANES)] = jnp.where(
                keep, idv - seg_base + region, region + SPT)
        pltpu.sync_copy(rows_v, acc_sh.at[lidx_v], add=True)
        return carry

    lax.fori_loop(0, nchunks, chunk, 0)

    pltpu.sync_copy(acc_sh.at[pl.ds(region, SPT)],
                    out_hbm.at[pl.ds(seg_base, SPT)])


@functools.partial(
    pl.kernel,
    out_type=jax.ShapeDtypeStruct((S, D), jnp.float32),
    mesh=plsc.VectorSubcoreMesh(core_axis_name="c", subcore_axis_name="s"),
    scratch_types=[
        pltpu.VMEM((C, D), jnp.float32),
        pltpu.VMEM((C,), jnp.int32),
        pltpu.VMEM((C,), jnp.int32),
        pltpu.VMEM((1, 16), jnp.int32),
        pltpu.MemorySpace.VMEM_SHARED((NS * ACC_ROWS, D), jnp.float32),
    ],
)
def _segment_sum_sc(nodes_hbm, batch_hbm, bounds_hbm, out_hbm,
                    rows_v, ids_v, lidx_v, bnd_v, acc_sh):
    _tile_body(nodes_hbm, batch_hbm, bounds_hbm, out_hbm,
               rows_v, ids_v, lidx_v, bnd_v, acc_sh)


def kernel(node_features, batch, ptr):
    edges = jnp.arange(0, S + 1, SPT, dtype=jnp.int32)
    starts = jnp.searchsorted(batch, edges, side="left").astype(jnp.int32)
    bounds = jnp.zeros((NW, 16), jnp.int32)
    bounds = bounds.at[:, 0].set(starts[:-1]).at[:, 1].set(starts[1:])
    return _segment_sum_sc(node_features, batch, bounds)


# speedup vs baseline: 5.9412x; 1.5536x over previous
import functools

import jax
import jax.numpy as jnp
from jax import lax
from jax.experimental import pallas as pl
from jax.experimental.pallas import tpu as pltpu
import jax.experimental.pallas.tpu_sc as plsc

N = 320000
D = 128
S = 2048
NC = 2
NS = 16
NW = NC * NS
SPT = S // NW
C = 128
NBUF = 4
ACC_ROWS = SPT + 8
LANES = 16


def _tile_body(nodes_hbm, batch_hbm, bounds_hbm, out_hbm,
               rows_v, ids_v, lidx_v, bnd_v, acc_sh, *sems):
    rsems = sems[:NBUF]
    isems = sems[NBUF:]
    sid = lax.axis_index("s")
    wid = sid * NC + lax.axis_index("c")
    seg_base = wid * SPT
    region = sid * ACC_ROWS

    pltpu.sync_copy(bounds_hbm.at[pl.ds(wid, 1)], bnd_v)
    lanes = lax.iota(jnp.int32, 16)
    bvec = bnd_v[0, :]
    start = bvec[0]
    end = bvec[1]

    zz = jnp.zeros((LANES,), jnp.float32)

    def zero_row(i, carry):
        for j in range(D // LANES):
            rows_v[0, i, pl.ds(j * LANES, LANES)] = zz
        return carry

    lax.fori_loop(0, ACC_ROWS, zero_row, 0)
    pltpu.sync_copy(rows_v.at[0].at[pl.ds(0, ACC_ROWS)],
                    acc_sh.at[pl.ds(region, ACC_ROWS)])

    astart = jnp.bitwise_and(start, jnp.int32(-8))
    nchunks = lax.shift_right_arithmetic(end - astart + (C - 1), 7)
    nouter = lax.shift_right_arithmetic(nchunks + (NBUF - 1), 2)

    def chunk_base(k):
        return pl.multiple_of(jnp.minimum(astart + k * C, N - C), 8)

    def fetch(k, b):
        base = chunk_base(k)
        pltpu.async_copy(batch_hbm.at[pl.ds(base, C)], ids_v.at[b], isems[b])
        pltpu.async_copy(nodes_hbm.at[pl.ds(base, C)], rows_v.at[b], rsems[b])

    def wait(k, b):
        base = chunk_base(k)
        pltpu.make_async_copy(
            batch_hbm.at[pl.ds(base, C)], ids_v.at[b], isems[b]).wait()
        pltpu.make_async_copy(
            nodes_hbm.at[pl.ds(base, C)], rows_v.at[b], rsems[b]).wait()

    for b in range(NBUF):
        fetch(jnp.int32(b), b)

    def outer(k0, carry):
        for b in range(NBUF):
            k = k0 * NBUF + b
            nominal = astart + k * C
            lo = jnp.maximum(start, nominal)
            hi = jnp.minimum(end, nominal + C)
            base = chunk_base(k)
            wait(k, b)
            for g in range(C // LANES):
                rg = base + (g * LANES) + lanes
                idv = ids_v[b, pl.ds(g * LANES, LANES)]
                keep = ((rg >= lo) & (rg < hi)
                        & (idv >= seg_base) & (idv < seg_base + SPT))
                lidx_v[b, pl.ds(g * LANES, LANES)] = jnp.where(
                    keep, idv - seg_base + region, region + SPT)
            pltpu.sync_copy(rows_v.at[b], acc_sh.at[lidx_v.at[b]], add=True)
            fetch(k + NBUF, b)
        return carry

    lax.fori_loop(0, nouter, outer, 0)

    for b in range(NBUF):
        wait(nouter * NBUF + b, b)

    pltpu.sync_copy(acc_sh.at[pl.ds(region, SPT)],
                    out_hbm.at[pl.ds(seg_base, SPT)])


@functools.partial(
    pl.kernel,
    out_type=jax.ShapeDtypeStruct((S, D), jnp.float32),
    mesh=plsc.VectorSubcoreMesh(core_axis_name="c", subcore_axis_name="s"),
    scratch_types=[
        pltpu.VMEM((NBUF, C, D), jnp.float32),
        pltpu.VMEM((NBUF, C), jnp.int32),
        pltpu.VMEM((NBUF, C), jnp.int32),
        pltpu.VMEM((1, 16), jnp.int32),
        pltpu.MemorySpace.VMEM_SHARED((NS * ACC_ROWS, D), jnp.float32),
    ] + [pltpu.SemaphoreType.DMA] * (2 * NBUF),
)
def _segment_sum_sc(nodes_hbm, batch_hbm, bounds_hbm, out_hbm,
                    rows_v, ids_v, lidx_v, bnd_v, acc_sh, *sems):
    _tile_body(nodes_hbm, batch_hbm, bounds_hbm, out_hbm,
               rows_v, ids_v, lidx_v, bnd_v, acc_sh, *sems)


def kernel(node_features, batch, ptr):
    edges = jnp.arange(0, S + 1, SPT, dtype=jnp.int32)
    starts = jnp.searchsorted(batch, edges, side="left").astype(jnp.int32)
    bounds = jnp.zeros((NW, 16), jnp.int32)
    bounds = bounds.at[:, 0].set(starts[:-1]).at[:, 1].set(starts[1:])
    return _segment_sum_sc(node_features, batch, bounds)


# speedup vs baseline: 7.7033x; 1.2966x over previous
import functools

import jax
import jax.numpy as jnp
from jax import lax
from jax.experimental import pallas as pl
from jax.experimental.pallas import tpu as pltpu
import jax.experimental.pallas.tpu_sc as plsc

N = 320000
D = 128
S = 2048
NC = 2
NS = 16
NW = NC * NS
RPT = N // NW
C = 80
NCHUNKS = RPT // C
NBUF = 5
NOUTER = NCHUNKS // NBUF
SROWS = S // NS
LANES = 16


def _tile_body(nodes_hbm, batch_hbm, pout_hbm,
               rows_v, ids_v, zbuf_v, acc_sh, *sems):
    rsems = sems[:NBUF]
    isems = sems[NBUF:]
    sid = lax.axis_index("s")
    cid = lax.axis_index("c")
    row0 = (cid * NS + sid) * RPT

    def chunk_base(k):
        return pl.multiple_of(jnp.minimum(row0 + k * C, N - C), 8)

    def fetch(k, b):
        base = chunk_base(k)
        pltpu.async_copy(batch_hbm.at[pl.ds(base, C)], ids_v.at[b], isems[b])
        pltpu.async_copy(nodes_hbm.at[pl.ds(base, C)], rows_v.at[b], rsems[b])

    def wait(k, b):
        base = chunk_base(k)
        pltpu.make_async_copy(
            batch_hbm.at[pl.ds(base, C)], ids_v.at[b], isems[b]).wait()
        pltpu.make_async_copy(
            nodes_hbm.at[pl.ds(base, C)], rows_v.at[b], rsems[b]).wait()

    for b in range(NBUF):
        fetch(jnp.int32(b), b)

    zz = jnp.zeros((LANES,), jnp.float32)

    def zero_row(i, carry):
        for j in range(D // LANES):
            zbuf_v[i, pl.ds(j * LANES, LANES)] = zz
        return carry

    lax.fori_loop(0, SROWS, zero_row, 0)
    pltpu.sync_copy(zbuf_v, acc_sh.at[pl.ds(sid * SROWS, SROWS)])
    plsc.subcore_barrier()

    def outer(k0, carry):
        for b in range(NBUF):
            k = k0 * NBUF + b
            wait(k, b)
            pltpu.sync_copy(rows_v.at[b], acc_sh.at[ids_v.at[b]], add=True)
            fetch(k + NBUF, b)
        return carry

    lax.fori_loop(0, NOUTER, outer, 0)

    for b in range(NBUF):
        wait(NCHUNKS + b, b)

    plsc.subcore_barrier()
    pltpu.sync_copy(acc_sh.at[pl.ds(sid * SROWS, SROWS)],
                    pout_hbm.at[cid].at[pl.ds(sid * SROWS, SROWS)])


@functools.partial(
    pl.kernel,
    out_type=jax.ShapeDtypeStruct((NC, S, D), jnp.float32),
    mesh=plsc.VectorSubcoreMesh(core_axis_name="c", subcore_axis_name="s"),
    scratch_types=[
        pltpu.VMEM((NBUF, C, D), jnp.float32),
        pltpu.VMEM((NBUF, C), jnp.int32),
        pltpu.VMEM((SROWS, D), jnp.float32),
        pltpu.MemorySpace.VMEM_SHARED((S, D), jnp.float32),
    ] + [pltpu.SemaphoreType.DMA] * (2 * NBUF),
)
def _segment_sum_sc(nodes_hbm, batch_hbm, pout_hbm,
                    rows_v, ids_v, zbuf_v, acc_sh, *sems):
    _tile_body(nodes_hbm, batch_hbm, pout_hbm,
               rows_v, ids_v, zbuf_v, acc_sh, *sems)


def _combine_body(p_ref, o_ref):
    o_ref[...] = p_ref[0] + p_ref[1]


def _combine(partials):
    blk = 256
    return pl.pallas_call(
        _combine_body,
        grid=(S // blk,),
        in_specs=[pl.BlockSpec((NC, blk, D), lambda i: (0, i, 0))],
        out_specs=pl.BlockSpec((blk, D), lambda i: (i, 0)),
        out_shape=jax.ShapeDtypeStruct((S, D), jnp.float32),
    )(partials)


def kernel(node_features, batch, ptr):
    partials = _segment_sum_sc(node_features, batch)
    return _combine(partials)
